# baseline (speedup 1.0000x reference)
import functools

import jax
import jax.numpy as jnp
from jax.experimental import pallas as pl
from jax.experimental.pallas import tpu as pltpu

_B = 4096
_N = 128
_K = 64
_ROWS = 8
_U_ITERS = 20
_Z_ITERS = 9


def _spike_sum(z, lo, hi):
    kf = jnp.float32(_K)
    for _ in range(_Z_ITERS):
        mid = 0.5 * (lo + hi)
        cnt = jnp.sum((z <= mid[:, None, :]).astype(jnp.float32), axis=1)
        ge = cnt >= kf
        hi = jnp.where(ge, mid, hi)
        lo = jnp.where(ge, lo, mid)
    t = hi[:, None, :]
    lt = z < t
    cnt_lt = jnp.sum(lt.astype(jnp.float32), axis=1)
    s_lt = jnp.sum(jnp.where(lt, z, 0.0), axis=1)
    return (s_lt + (kf - cnt_lt) * hi) * (1.0 / _K)


def _body(x_ref, w_ref, o_ref):
    x = x_ref[...]
    w = w_ref[...]
    a = jnp.maximum(3.0 + x, 0.0)
    b = jnp.maximum(3.0 - x, 0.0)
    p = jnp.maximum(3.0 + w, 0.0)
    m = jnp.maximum(3.0 - w, 0.0)
    u = jnp.concatenate([a, b], axis=1)

    u_hi = jnp.max(u, axis=1)
    u_lo = jnp.zeros_like(u_hi)
    kf = jnp.float32(_K)
    for _ in range(_U_ITERS):
        mid = 0.5 * (u_lo + u_hi)
        cnt = jnp.sum((u <= mid[:, None]).astype(jnp.float32), axis=1)
        ge = cnt >= kf
        u_hi = jnp.where(ge, mid, u_hi)
        u_lo = jnp.where(ge, u_lo, mid)

    v_p = jnp.concatenate([p, m], axis=0)
    v_m = jnp.concatenate([m, p], axis=0)
    v_min = jnp.min(v_p, axis=0)[None, :]
    v_max = jnp.max(v_p, axis=0)[None, :]
    lo0 = u_lo[:, None] + v_min
    hi0 = u_hi[:, None] + v_max

    uu = u[:, :, None]
    s_plus = _spike_sum(uu + v_p[None, :, :], lo0, hi0)
    s_minus = _spike_sum(uu + v_m[None, :, :], lo0, hi0)
    o_ref[...] = s_plus - s_minus


@jax.jit
def kernel(inputp, W):
    grid = _B // _ROWS
    return pl.pallas_call(
        _body,
        grid=(grid,),
        in_specs=[
            pl.BlockSpec((_ROWS, _N), lambda i: (i, 0)),
            pl.BlockSpec((_N, _N), lambda i: (0, 0)),
        ],
        out_specs=pl.BlockSpec((_ROWS, _N), lambda i: (i, 0)),
        out_shape=jax.ShapeDtypeStruct((_B, _N), jnp.float32),
    )(inputp, W)


# speedup vs baseline: 1.3183x; 1.3183x over previous
import functools

import jax
import jax.numpy as jnp
from jax.experimental import pallas as pl
from jax.experimental.pallas import tpu as pltpu

_B = 4096
_N = 128
_K = 64
_ROWS = 32
_U_ITERS = 14
_Z_ITERS = 9


def _spike_sum(z, lo, hi):
    kf = jnp.float32(_K)
    for _ in range(_Z_ITERS):
        mid = 0.5 * (lo + hi)
        cnt = jnp.sum((z <= mid[:, None, :]).astype(jnp.float32), axis=1)
        ge = cnt >= kf
        hi = jnp.where(ge, mid, hi)
        lo = jnp.where(ge, lo, mid)
    t = hi[:, None, :]
    lt = z < t
    cnt_lt = jnp.sum(lt.astype(jnp.float32), axis=1)
    s_lt = jnp.sum(jnp.where(lt, z, 0.0), axis=1)
    return (s_lt + (kf - cnt_lt) * hi) * (1.0 / _K)


def _body(x_ref, w_ref, o_ref):
    x = x_ref[...]
    w = w_ref[...]
    a = jnp.maximum(3.0 + x, 0.0)
    b = jnp.maximum(3.0 - x, 0.0)
    p = jnp.maximum(3.0 + w, 0.0)
    m = jnp.maximum(3.0 - w, 0.0)
    u = jnp.concatenate([a, b], axis=1)

    u_hi = jnp.max(u, axis=1)
    u_lo = jnp.zeros_like(u_hi)
    kf = jnp.float32(_K)
    for _ in range(_U_ITERS):
        mid = 0.5 * (u_lo + u_hi)
        cnt = jnp.sum((u <= mid[:, None]).astype(jnp.float32), axis=1)
        ge = cnt >= kf
        u_hi = jnp.where(ge, mid, u_hi)
        u_lo = jnp.where(ge, u_lo, mid)

    v_p = jnp.concatenate([p, m], axis=0)
    v_m = jnp.concatenate([m, p], axis=0)
    v_min = jnp.min(v_p, axis=0)[None, :]
    v_max = jnp.max(v_p, axis=0)[None, :]
    lo0 = u_lo[:, None] + v_min
    hi0 = u_hi[:, None] + v_max

    uu = u[:, :, None]
    s_plus = _spike_sum(uu + v_p[None, :, :], lo0, hi0)
    s_minus = _spike_sum(uu + v_m[None, :, :], lo0, hi0)
    o_ref[...] = s_plus - s_minus


@jax.jit
def kernel(inputp, W):
    grid = _B // _ROWS
    return pl.pallas_call(
        _body,
        grid=(grid,),
        in_specs=[
            pl.BlockSpec((_ROWS, _N), lambda i: (i, 0)),
            pl.BlockSpec((_N, _N), lambda i: (0, 0)),
        ],
        out_specs=pl.BlockSpec((_ROWS, _N), lambda i: (i, 0)),
        out_shape=jax.ShapeDtypeStruct((_B, _N), jnp.float32),
    )(inputp, W)


# speedup vs baseline: 1.8388x; 1.3948x over previous
import functools

import jax
import jax.numpy as jnp
from jax.experimental import pallas as pl
from jax.experimental.pallas import tpu as pltpu

_B = 4096
_N = 128
_K = 64
_ROWS = 32
_U_ITERS = 14
_Z_ITERS = 7


def _spike_sum(z, lo, hi):
    kf = jnp.float32(_K)
    for _ in range(_Z_ITERS):
        mid = 0.5 * (lo + hi)
        cnt = jnp.sum((z <= mid[:, None, :]).astype(jnp.float32), axis=1)
        ge = cnt >= kf
        hi = jnp.where(ge, mid, hi)
        lo = jnp.where(ge, lo, mid)
    t = hi[:, None, :]
    s_min = jnp.sum(jnp.minimum(z, t), axis=1)
    return (s_min - jnp.float32(2 * _N - _K) * hi) * (1.0 / _K)


def _body(x_ref, w_ref, o_ref):
    x = x_ref[...]
    w = w_ref[...]
    a = jnp.maximum(3.0 + x, 0.0)
    b = jnp.maximum(3.0 - x, 0.0)
    p = jnp.maximum(3.0 + w, 0.0)
    m = jnp.maximum(3.0 - w, 0.0)
    u = jnp.concatenate([a, b], axis=1)

    u_hi = jnp.max(u, axis=1)
    u_lo = jnp.zeros_like(u_hi)
    kf = jnp.float32(_K)
    for _ in range(_U_ITERS):
        mid = 0.5 * (u_lo + u_hi)
        cnt = jnp.sum((u <= mid[:, None]).astype(jnp.float32), axis=1)
        ge = cnt >= kf
        u_hi = jnp.where(ge, mid, u_hi)
        u_lo = jnp.where(ge, u_lo, mid)

    v_p = jnp.concatenate([p, m], axis=0)
    v_m = jnp.concatenate([m, p], axis=0)
    v_min = jnp.min(v_p, axis=0)[None, :]
    v_max = jnp.max(v_p, axis=0)[None, :]
    lo0 = u_lo[:, None] + v_min
    hi0 = u_hi[:, None] + v_max

    uu = u[:, :, None]
    s_plus = _spike_sum(uu + v_p[None, :, :], lo0, hi0)
    s_minus = _spike_sum(uu + v_m[None, :, :], lo0, hi0)
    o_ref[...] = s_plus - s_minus


@jax.jit
def kernel(inputp, W):
    grid = _B // _ROWS
    return pl.pallas_call(
        _body,
        grid=(grid,),
        in_specs=[
            pl.BlockSpec((_ROWS, _N), lambda i: (i, 0)),
            pl.BlockSpec((_N, _N), lambda i: (0, 0)),
        ],
        out_specs=pl.BlockSpec((_ROWS, _N), lambda i: (i, 0)),
        out_shape=jax.ShapeDtypeStruct((_B, _N), jnp.float32),
    )(inputp, W)
